# baseline (speedup 1.0000x reference)
import dataclasses
import functools

import jax
import jax.numpy as jnp
from jax import lax
from jax.experimental import pallas as pl
from jax.experimental.pallas import tpu as pltpu
from jax.experimental.pallas import tpu_sc as plsc

N = 10000
E = 320000
FIN = 128
FQK = 128
FV = 128
H = 8
FH = 16
NTILES = 32
EPT = E // NTILES
EB = 32
STEPS = EPT // EB
TAIL = EPT - (EPT // EB) * EB
NP = 10240
ND = NP // 8
RPT = NP // 16
DPT = ND // 16
ZB = 16


def _proj_body(x_ref, w_ref, q_ref, kv_ref):
    p = lax.dot_general(x_ref[...], w_ref[...], (((1,), (1,)), ((), ())),
                        preferred_element_type=jnp.float32)
    q_ref[...] = p[:, :FQK] * (FH ** -0.5)
    kv_ref[...] = p[:, FQK:]


def _project(x, W):
    blk = 1000
    grid = (N // blk,)
    return pl.pallas_call(
        _proj_body,
        grid=grid,
        in_specs=[
            pl.BlockSpec((blk, FIN), lambda i: (i, 0)),
            pl.BlockSpec((2 * FQK + FV, FIN), lambda i: (0, 0)),
        ],
        out_specs=[
            pl.BlockSpec((blk, FQK), lambda i: (i, 0)),
            pl.BlockSpec((blk, FQK + FV), lambda i: (i, 0)),
        ],
        out_shape=[
            jax.ShapeDtypeStruct((N, FQK), jnp.float32),
            jax.ShapeDtypeStruct((N, FQK + FV), jnp.float32),
        ],
    )(x, W)


def _sc_body(q_hbm, kv_hbm, src_hbm, dst_hbm, num_hbm, den_hbm,
             srcv0, dstv0, didx0, qv0, kvv0,
             srcv1, dstv1, didx1, qv1, kvv1,
             srcv_t, dstv_t, didx_t,
             wvv, dnv, zv, acc_n, acc_d,
             sem_i0, sem_i1, sem_g0, sem_g1, sem_z):
    cid = lax.axis_index("c")
    sid = lax.axis_index("s")
    wid = cid * 16 + sid
    lanes = lax.broadcasted_iota(jnp.int32, (16,), 0)
    zero16 = jnp.zeros((16,), jnp.float32)
    S = (srcv0, srcv1)
    D = (dstv0, dstv1)
    X = (didx0, didx1)
    Q = (qv0, qv1)
    K = (kvv0, kvv1)
    SI = (sem_i0, sem_i1)
    SG = (sem_g0, sem_g1)

    @pl.loop(0, ZB)
    def _(i):
        for j in range(FV // 16):
            zv[i, pl.ds(16 * j, 16)] = zero16

    ztargets = [acc_n.at[pl.ds(sid * RPT + r * ZB, ZB)]
                for r in range(RPT // ZB)]
    ztargets += [acc_d.at[pl.ds(sid * DPT + r * ZB, ZB)]
                 for r in range(DPT // ZB)]
    for wave in range(0, len(ztargets), 15):
        hs = [pltpu.async_copy(zv, t, sem_z) for t in ztargets[wave:wave + 15]]
        for hh in hs:
            hh.wait()
    plsc.subcore_barrier()

    tb = wid * EPT

    def issue_idx(b, step):
        base = tb + step * EB
        pltpu.async_copy(src_hbm.at[pl.ds(base, EB)], S[b], SI[b])
        pltpu.async_copy(dst_hbm.at[pl.ds(base, EB)], D[b], SI[b])

    def wait_idx(b):
        pltpu.make_async_copy(src_hbm.at[pl.ds(0, EB)], S[b], SI[b]).wait()
        pltpu.make_async_copy(dst_hbm.at[pl.ds(0, EB)], D[b], SI[b]).wait()

    def issue_gather(b):
        pltpu.async_copy(q_hbm.at[S[b]], Q[b], SG[b])
        pltpu.async_copy(kv_hbm.at[D[b]], K[b], SG[b])

    def wait_gather(b):
        pltpu.make_async_copy(q_hbm.at[S[b]], Q[b], SG[b]).wait()
        pltpu.make_async_copy(kv_hbm.at[D[b]], K[b], SG[b]).wait()

    def edge_block(sv_ref, qr, kr, nedge, didx_ref):
        @pl.loop(0, nedge // 16)
        def _(c):
            sv = sv_ref[pl.ds(c * 16, 16)]
            didx_ref[pl.ds(c * 16, 16)] = lax.shift_right_logical(sv, 3)
            grpv = sv & 7
            for l in range(16):
                e = c * 16 + l
                grp = grpv[l]
                den = zero16
                for h in range(H):
                    qh = qr[e, pl.ds(16 * h, 16)]
                    kh = kr[e, pl.ds(16 * h, 16)]
                    vh = kr[e, pl.ds(FQK + 16 * h, 16)]
                    s = jnp.sum(qh * kh)
                    w = jnp.exp(lax.broadcast(s, (16,)))
                    wvv[e, pl.ds(16 * h, 16)] = w * vh
                    den = den + jnp.where(lanes == h, w, 0.0)
                for g in range(8):
                    sel = lax.broadcast(grp == g, (16,))
                    dnv[e, pl.ds(16 * g, 16)] = lax.select(sel, den, zero16)

    issue_idx(0, 0)
    issue_idx(1, 1)
    wait_idx(0)
    issue_gather(0)

    @pl.loop(0, STEPS // 2)
    def _(it):
        for b in (0, 1):
            i = it * 2 + b
            nb = 1 - b

            @pl.when(i + 1 < STEPS)
            def _():
                wait_idx(nb)
                issue_gather(nb)

            wait_gather(b)
            edge_block(S[b], Q[b], K[b], EB, X[b])
            pltpu.sync_copy(wvv, acc_n.at[S[b]], add=True)
            pltpu.sync_copy(dnv, acc_d.at[X[b]], add=True)

            @pl.when(i + 2 < STEPS)
            def _():
                issue_idx(b, i + 2)

    base_t = tb + STEPS * EB
    pltpu.sync_copy(src_hbm.at[pl.ds(base_t, TAIL)], srcv_t)
    pltpu.sync_copy(dst_hbm.at[pl.ds(base_t, TAIL)], dstv_t)
    pltpu.sync_copy(q_hbm.at[srcv_t], qv0.at[pl.ds(0, TAIL)])
    pltpu.sync_copy(kv_hbm.at[dstv_t], kvv0.at[pl.ds(0, TAIL)])
    edge_block(srcv_t, qv0, kvv0, TAIL, didx_t)
    pltpu.sync_copy(wvv.at[pl.ds(0, TAIL)], acc_n.at[srcv_t], add=True)
    pltpu.sync_copy(dnv.at[pl.ds(0, TAIL)], acc_d.at[didx_t], add=True)

    plsc.subcore_barrier()
    WB = EB
    for r in range(RPT // WB):
        pltpu.sync_copy(acc_n.at[pl.ds(sid * RPT + r * WB, WB)], wvv)
        pltpu.sync_copy(wvv, num_hbm.at[cid, pl.ds(sid * RPT + r * WB, WB)])
    for r in range(DPT // TAIL):
        pltpu.sync_copy(acc_d.at[pl.ds(sid * DPT + r * TAIL, TAIL)],
                        wvv.at[pl.ds(0, TAIL)])
        pltpu.sync_copy(wvv.at[pl.ds(0, TAIL)],
                        den_hbm.at[cid, pl.ds(sid * DPT + r * TAIL, TAIL)])


def _sc_attend(q_tbl, kv_tbl, src, dst):
    mesh = plsc.VectorSubcoreMesh(core_axis_name="c", subcore_axis_name="s")
    cp = pltpu.CompilerParams()
    if "needs_layout_passes" in pltpu.CompilerParams.__dataclass_fields__:
        cp = dataclasses.replace(cp, needs_layout_passes=False)
    dbuf = [
        pltpu.VMEM((EB,), jnp.int32),
        pltpu.VMEM((EB,), jnp.int32),
        pltpu.VMEM((EB,), jnp.int32),
        pltpu.VMEM((EB, FQK), jnp.float32),
        pltpu.VMEM((EB, FQK + FV), jnp.float32),
    ]
    fn = pl.kernel(
        _sc_body,
        compiler_params=cp,
        out_type=[
            jax.ShapeDtypeStruct((2, NP, FV), jnp.float32),
            jax.ShapeDtypeStruct((2, ND, 128), jnp.float32),
        ],
        mesh=mesh,
        scratch_types=dbuf + dbuf + [
            pltpu.VMEM((TAIL,), jnp.int32),
            pltpu.VMEM((TAIL,), jnp.int32),
            pltpu.VMEM((TAIL,), jnp.int32),
            pltpu.VMEM((EB, FV), jnp.float32),
            pltpu.VMEM((EB, 128), jnp.float32),
            pltpu.VMEM((ZB, 128), jnp.float32),
            pltpu.VMEM_SHARED((NP, FV), jnp.float32),
            pltpu.VMEM_SHARED((ND, 128), jnp.float32),
            pltpu.SemaphoreType.DMA,
            pltpu.SemaphoreType.DMA,
            pltpu.SemaphoreType.DMA,
            pltpu.SemaphoreType.DMA,
            pltpu.SemaphoreType.DMA,
        ],
    )
    return fn(q_tbl, kv_tbl, src, dst)


def _comb_body(num_ref, den_ref, o_ref):
    num = num_ref[0] + num_ref[1]
    den16 = den_ref[0] + den_ref[1]
    col = lax.broadcasted_iota(jnp.int32, (16, FV), 1) // FH
    row = lax.broadcasted_iota(jnp.int32, (16, FV), 0)
    ex = (col == row).astype(jnp.float32)
    rep = lax.dot_general(den16, ex, (((1,), (0,)), ((), ())),
                          preferred_element_type=jnp.float32)
    o_ref[...] = jnp.where(rep > 0, num / rep, 0.0)


def _combine(nd_num, nd_den16):
    blk = 1000
    return pl.pallas_call(
        _comb_body,
        grid=(N // blk,),
        in_specs=[
            pl.BlockSpec((2, blk, FV), lambda i: (0, i, 0)),
            pl.BlockSpec((2, blk, 16), lambda i: (0, i, 0)),
        ],
        out_specs=pl.BlockSpec((blk, FV), lambda i: (i, 0)),
        out_shape=jax.ShapeDtypeStruct((N, FV), jnp.float32),
    )(nd_num, nd_den16)


def kernel(x, batch, ei, W):
    del batch
    q_tbl, kv_tbl = _project(x, W)
    nd_num, nd_den = _sc_attend(q_tbl, kv_tbl, ei[0], ei[1])
    return _combine(nd_num, nd_den.reshape(2, NP, 16))


# speedup vs baseline: 2.8764x; 2.8764x over previous
import dataclasses
import functools

import jax
import jax.numpy as jnp
from jax import lax
from jax.experimental import pallas as pl
from jax.experimental.pallas import tpu as pltpu
from jax.experimental.pallas import tpu_sc as plsc

N = 10000
E = 320000
FIN = 128
FQK = 128
FV = 128
H = 8
FH = 16
NTILES = 32
EPT = E // NTILES
NP = 10240
ND = NP // 8
RPT = NP // 16
DPT = ND // 16
ZB = 16

EA = 128
SA = EPT // EA
TA = EPT - SA * EA

EC = 64
SC = EPT // EC
TC = EPT - SC * EC


def _compiler_params():
    cp = pltpu.CompilerParams()
    if "needs_layout_passes" in pltpu.CompilerParams.__dataclass_fields__:
        cp = dataclasses.replace(cp, needs_layout_passes=False)
    return cp


def _mesh():
    return plsc.VectorSubcoreMesh(core_axis_name="c", subcore_axis_name="s")


def _proj_body(x_ref, w_ref, q_ref, kv_ref):
    p = lax.dot_general(x_ref[...], w_ref[...], (((1,), (1,)), ((), ())),
                        preferred_element_type=jnp.float32)
    q_ref[...] = p[:, :FQK] * (FH ** -0.5)
    kv_ref[...] = p[:, FQK:]


def _project(x, W):
    blk = 1000
    return pl.pallas_call(
        _proj_body,
        grid=(N // blk,),
        in_specs=[
            pl.BlockSpec((blk, FIN), lambda i: (i, 0)),
            pl.BlockSpec((2 * FQK + FV, FIN), lambda i: (0, 0)),
        ],
        out_specs=[
            pl.BlockSpec((blk, FQK), lambda i: (i, 0)),
            pl.BlockSpec((blk, FQK + FV), lambda i: (i, 0)),
        ],
        out_shape=[
            jax.ShapeDtypeStruct((N, FQK), jnp.float32),
            jax.ShapeDtypeStruct((N, FQK + FV), jnp.float32),
        ],
    )(x, W)


def _sc_gather_body(q_hbm, kv_hbm, src_hbm, dst_hbm, qs_hbm, kvs_hbm,
                    s0, d0, q0, k0, s1, d1, q1, k1, st, dt,
                    i0, i1, g0, g1, w0, w1):
    cid = lax.axis_index("c")
    sid = lax.axis_index("s")
    tb = (cid * 16 + sid) * EPT
    S = (s0, s1)
    D = (d0, d1)
    Q = (q0, q1)
    K = (k0, k1)
    SI = (i0, i1)
    SG = (g0, g1)
    SW = (w0, w1)

    def issue_idx(b, step):
        base = tb + step * EA
        pltpu.async_copy(src_hbm.at[pl.ds(base, EA)], S[b], SI[b])
        pltpu.async_copy(dst_hbm.at[pl.ds(base, EA)], D[b], SI[b])

    def wait_idx(b):
        pltpu.make_async_copy(src_hbm.at[pl.ds(0, EA)], S[b], SI[b]).wait()
        pltpu.make_async_copy(dst_hbm.at[pl.ds(0, EA)], D[b], SI[b]).wait()

    def issue_gather(b):
        pltpu.async_copy(q_hbm.at[S[b]], Q[b], SG[b])
        pltpu.async_copy(kv_hbm.at[D[b]], K[b], SG[b])

    def wait_gather(b):
        pltpu.make_async_copy(q_hbm.at[S[b]], Q[b], SG[b]).wait()
        pltpu.make_async_copy(kv_hbm.at[D[b]], K[b], SG[b]).wait()

    def issue_write(b, step):
        base = tb + step * EA
        pltpu.async_copy(Q[b], qs_hbm.at[pl.ds(base, EA)], SW[b])
        pltpu.async_copy(K[b], kvs_hbm.at[pl.ds(base, EA)], SW[b])

    def wait_write(b):
        pltpu.make_async_copy(Q[b], qs_hbm.at[pl.ds(0, EA)], SW[b]).wait()
        pltpu.make_async_copy(K[b], kvs_hbm.at[pl.ds(0, EA)], SW[b]).wait()

    issue_idx(0, 0)
    issue_idx(1, 1)
    wait_idx(0)
    issue_gather(0)

    @pl.loop(0, SA // 2)
    def _(it):
        for b in (0, 1):
            i = it * 2 + b
            nb = 1 - b

            @pl.when(i + 1 < SA)
            def _():
                wait_idx(nb)

            @pl.when(jnp.logical_and(i + 1 < SA, i >= 1))
            def _():
                wait_write(nb)

            @pl.when(i + 1 < SA)
            def _():
                issue_gather(nb)

            wait_gather(b)
            issue_write(b, i)

            @pl.when(i + 2 < SA)
            def _():
                issue_idx(b, i + 2)

    wait_write(0)
    wait_write(1)

    base_t = tb + SA * EA
    pltpu.sync_copy(src_hbm.at[pl.ds(base_t, TA)], st)
    pltpu.sync_copy(dst_hbm.at[pl.ds(base_t, TA)], dt)
    pltpu.sync_copy(q_hbm.at[st], q0.at[pl.ds(0, TA)])
    pltpu.sync_copy(kv_hbm.at[dt], k0.at[pl.ds(0, TA)])
    pltpu.sync_copy(q0.at[pl.ds(0, TA)], qs_hbm.at[pl.ds(base_t, TA)])
    pltpu.sync_copy(k0.at[pl.ds(0, TA)], kvs_hbm.at[pl.ds(base_t, TA)])


def _sc_gather(q_tbl, kv_tbl, src, dst):
    dbuf = [
        pltpu.VMEM((EA,), jnp.int32),
        pltpu.VMEM((EA,), jnp.int32),
        pltpu.VMEM((EA, FQK), jnp.float32),
        pltpu.VMEM((EA, FQK + FV), jnp.float32),
    ]
    fn = pl.kernel(
        _sc_gather_body,
        compiler_params=_compiler_params(),
        out_type=[
            jax.ShapeDtypeStruct((E, FQK), jnp.float32),
            jax.ShapeDtypeStruct((E, FQK + FV), jnp.float32),
        ],
        mesh=_mesh(),
        scratch_types=dbuf + dbuf + [
            pltpu.VMEM((TA,), jnp.int32),
            pltpu.VMEM((TA,), jnp.int32),
        ] + [pltpu.SemaphoreType.DMA] * 6,
    )
    return fn(q_tbl, kv_tbl, src, dst)


def _edge_body(src_ref, qs_ref, kvs_ref, wv_ref, dn_ref):
    blk = qs_ref.shape[0]
    qs = qs_ref[...]
    ks = kvs_ref[:, :FQK]
    vs = kvs_ref[:, FQK:]
    prod = qs * ks
    ch = lax.broadcasted_iota(jnp.int32, (FQK, H), 0) // FH
    hh = lax.broadcasted_iota(jnp.int32, (FQK, H), 1)
    sum16 = (ch == hh).astype(jnp.float32)
    hr = lax.broadcasted_iota(jnp.int32, (H, FV), 0)
    hc = lax.broadcasted_iota(jnp.int32, (H, FV), 1)
    expand = ((hc // FH) == hr).astype(jnp.float32)
    tile8 = ((hc & 15) == hr).astype(jnp.float32)

    aw = lax.dot_general(prod, sum16, (((1,), (0,)), ((), ())),
                         preferred_element_type=jnp.float32)
    w = jnp.exp(aw)
    wrep = lax.dot_general(w, expand, (((1,), (0,)), ((), ())),
                           preferred_element_type=jnp.float32)
    wv_ref[...] = wrep * vs
    wtile = lax.dot_general(w, tile8, (((1,), (0,)), ((), ())),
                            preferred_element_type=jnp.float32)
    grp = jnp.broadcast_to(src_ref[...] & 7, (blk, FV))
    lane16 = lax.broadcasted_iota(jnp.int32, (blk, FV), 1) // FH
    dn_ref[...] = jnp.where(lane16 == grp, wtile, 0.0)


def _edge_compute(src2, qs, kvs):
    blk = 2000
    return pl.pallas_call(
        _edge_body,
        grid=(E // blk,),
        in_specs=[
            pl.BlockSpec((blk, 1), lambda i: (i, 0)),
            pl.BlockSpec((blk, FQK), lambda i: (i, 0)),
            pl.BlockSpec((blk, FQK + FV), lambda i: (i, 0)),
        ],
        out_specs=[
            pl.BlockSpec((blk, FV), lambda i: (i, 0)),
            pl.BlockSpec((blk, 128), lambda i: (i, 0)),
        ],
        out_shape=[
            jax.ShapeDtypeStruct((E, FV), jnp.float32),
            jax.ShapeDtypeStruct((E, 128), jnp.float32),
        ],
    )(src2, qs, kvs)


def _sc_scatter_body(wv_hbm, dn_hbm, src_hbm, num_hbm, den_hbm,
                     s0, w0, n0, s1, w1, n1, st, didx, dt_, zv,
                     acc_n, acc_d, i0, i1, l0, l1, sz):
    cid = lax.axis_index("c")
    sid = lax.axis_index("s")
    tb = (cid * 16 + sid) * EPT
    zero16 = jnp.zeros((16,), jnp.float32)
    S = (s0, s1)
    Wb = (w0, w1)
    Nb = (n0, n1)
    SI = (i0, i1)
    SL = (l0, l1)

    @pl.loop(0, ZB)
    def _(i):
        for j in range(FV // 16):
            zv[i, pl.ds(16 * j, 16)] = zero16

    ztargets = [acc_n.at[pl.ds(sid * RPT + r * ZB, ZB)]
                for r in range(RPT // ZB)]
    ztargets += [acc_d.at[pl.ds(sid * DPT + r * ZB, ZB)]
                 for r in range(DPT // ZB)]
    for wave in range(0, len(ztargets), 15):
        hs = [pltpu.async_copy(zv, t, sz) for t in ztargets[wave:wave + 15]]
        for hh in hs:
            hh.wait()
    plsc.subcore_barrier()

    def issue_load(b, step):
        base = tb + step * EC
        pltpu.async_copy(src_hbm.at[pl.ds(base, EC)], S[b], SI[b])
        pltpu.async_copy(wv_hbm.at[pl.ds(base, EC)], Wb[b], SL[b])
        pltpu.async_copy(dn_hbm.at[pl.ds(base, EC)], Nb[b], SL[b])

    def wait_load(b):
        pltpu.make_async_copy(src_hbm.at[pl.ds(0, EC)], S[b], SI[b]).wait()
        pltpu.make_async_copy(wv_hbm.at[pl.ds(0, EC)], Wb[b], SL[b]).wait()
        pltpu.make_async_copy(dn_hbm.at[pl.ds(0, EC)], Nb[b], SL[b]).wait()

    issue_load(0, 0)
    issue_load(1, 1)

    @pl.loop(0, SC // 2)
    def _(it):
        for b in (0, 1):
            i = it * 2 + b
            wait_load(b)

            @pl.loop(0, EC // 16)
            def _(c):
                didx[pl.ds(c * 16, 16)] = lax.shift_right_logical(
                    S[b][pl.ds(c * 16, 16)], 3)

            pltpu.sync_copy(Wb[b], acc_n.at[S[b]], add=True)
            pltpu.sync_copy(Nb[b], acc_d.at[didx], add=True)

            @pl.when(i + 2 < SC)
            def _():
                issue_load(b, i + 2)

    base_t = tb + SC * EC
    pltpu.sync_copy(src_hbm.at[pl.ds(base_t, TC)], st)
    pltpu.sync_copy(wv_hbm.at[pl.ds(base_t, TC)], w0.at[pl.ds(0, TC)])
    pltpu.sync_copy(dn_hbm.at[pl.ds(base_t, TC)], n0.at[pl.ds(0, TC)])
    dt_[pl.ds(0, 16)] = lax.shift_right_logical(st[pl.ds(0, 16)], 3)
    pltpu.sync_copy(w0.at[pl.ds(0, TC)], acc_n.at[st], add=True)
    pltpu.sync_copy(n0.at[pl.ds(0, TC)], acc_d.at[dt_], add=True)

    plsc.subcore_barrier()
    for r in range(RPT // EC):
        pltpu.sync_copy(acc_n.at[pl.ds(sid * RPT + r * EC, EC)], w0)
        pltpu.sync_copy(w0, num_hbm.at[cid, pl.ds(sid * RPT + r * EC, EC)])
    for r in range(DPT // 16):
        pltpu.sync_copy(acc_d.at[pl.ds(sid * DPT + r * 16, 16)],
                        w0.at[pl.ds(0, 16)])
        pltpu.sync_copy(w0.at[pl.ds(0, 16)],
                        den_hbm.at[cid, pl.ds(sid * DPT + r * 16, 16)])


def _sc_scatter(wv, dn, src):
    dbuf = [
        pltpu.VMEM((EC,), jnp.int32),
        pltpu.VMEM((EC, FV), jnp.float32),
        pltpu.VMEM((EC, 128), jnp.float32),
    ]
    fn = pl.kernel(
        _sc_scatter_body,
        compiler_params=_compiler_params(),
        out_type=[
            jax.ShapeDtypeStruct((2, NP, FV), jnp.float32),
            jax.ShapeDtypeStruct((2, ND, 128), jnp.float32),
        ],
        mesh=_mesh(),
        scratch_types=dbuf + dbuf + [
            pltpu.VMEM((TC,), jnp.int32),
            pltpu.VMEM((EC,), jnp.int32),
            pltpu.VMEM((TC,), jnp.int32),
            pltpu.VMEM((ZB, 128), jnp.float32),
            pltpu.VMEM_SHARED((NP, FV), jnp.float32),
            pltpu.VMEM_SHARED((ND, 128), jnp.float32),
        ] + [pltpu.SemaphoreType.DMA] * 5,
    )
    return fn(wv, dn, src)


def _comb_body(num_ref, den_ref, o_ref):
    num = num_ref[0] + num_ref[1]
    den16 = den_ref[0] + den_ref[1]
    col = lax.broadcasted_iota(jnp.int32, (16, FV), 1) // FH
    row = lax.broadcasted_iota(jnp.int32, (16, FV), 0)
    ex = (col == row).astype(jnp.float32)
    rep = lax.dot_general(den16, ex, (((1,), (0,)), ((), ())),
                          preferred_element_type=jnp.float32)
    o_ref[...] = jnp.where(rep > 0, num / rep, 0.0)


def _combine(nd_num, nd_den16):
    blk = 1000
    return pl.pallas_call(
        _comb_body,
        grid=(N // blk,),
        in_specs=[
            pl.BlockSpec((2, blk, FV), lambda i: (0, i, 0)),
            pl.BlockSpec((2, blk, 16), lambda i: (0, i, 0)),
        ],
        out_specs=pl.BlockSpec((blk, FV), lambda i: (i, 0)),
        out_shape=jax.ShapeDtypeStruct((N, FV), jnp.float32),
    )(nd_num, nd_den16)


def kernel(x, batch, ei, W):
    del batch
    src = ei[0]
    dst = ei[1]
    q_tbl, kv_tbl = _project(x, W)
    qs, kvs = _sc_gather(q_tbl, kv_tbl, src, dst)
    wv, dn = _edge_compute(src.reshape(E, 1), qs, kvs)
    nd_num, nd_den = _sc_scatter(wv, dn, src)
    return _combine(nd_num, nd_den.reshape(2, NP, 16))
